# baseline (speedup 1.0000x reference)
import functools

import jax
import jax.numpy as jnp
from jax import lax
from jax.experimental import pallas as pl
from jax.experimental.pallas import tpu as pltpu
from jax.experimental.pallas import tpu_sc as plsc

CLS = 1_000_000
FEAT = 32
BATCH = 16384
NC = 2
NS = 16
L = 16
NW = NC * NS
B_T = BATCH // NS
SPLIT = 500_096
EDGE = 999_936
TAILD = 500_096
COLW = 500_224
CS = 31_232
CS_COVER = NS * CS
TBL = 500_736
REGION = TBL // NS
ZCH = REGION // 6
DMY = 500_160
W_SCALE = 2.0 * FEAT * CLS


def _body(xs_h, ys_h, cen_h, out_h,
          table, colbuf, zbuf, ysv, yloc, ylocb, ysct, yval,
          cntv, dacc, cgath, xst, tailv, partv, sem0, sem1, semg, semh):
    c = lax.axis_index("c")
    s = lax.axis_index("s")
    wid = s * NC + c
    sems = [sem0, sem1]
    iota16 = lax.iota(jnp.int32, L)
    cb = jnp.full((L,), c)

    def stage(f, buf):
        boff = buf * COLW
        so = pl.multiple_of(c * SPLIT + s * CS, 128)
        do = pl.multiple_of(boff + s * CS, 8)
        hs = [
            pltpu.async_copy(cen_h.at[f].at[pl.ds(so, CS)],
                             colbuf.at[pl.ds(do, CS)], sems[buf]),
            pltpu.async_copy(xs_h.at[f].at[pl.ds(s * B_T, B_T)],
                             xst.at[buf], sems[buf]),
        ]

        @pl.when(s == 0)
        def _():
            @pl.when(c == 0)
            def _():
                pltpu.sync_copy(cen_h.at[f].at[pl.ds(CS_COVER, 384)],
                                colbuf.at[pl.ds(boff + CS_COVER, 384)])
                pltpu.sync_copy(cen_h.at[f].at[pl.ds(EDGE, 64)], tailv)
                pltpu.sync_copy(tailv, colbuf.at[pl.ds(boff + TAILD, 64)])

            @pl.when(c == 1)
            def _():
                pltpu.sync_copy(cen_h.at[f].at[pl.ds(SPLIT + CS_COVER, 128)],
                                colbuf.at[pl.ds(boff + CS_COVER, 128)])

        return hs

    pending = stage(0, 0)
    stage(1, 1)

    zero16 = jnp.zeros((L,), jnp.float32)

    def zfill(i, carry):
        zbuf[pl.ds(i * L, L)] = zero16
        return carry

    lax.fori_loop(0, ZCH // L, zfill, 0)
    for k in range(REGION // ZCH):
        off = pl.multiple_of(s * REGION + k * ZCH, 8)
        pltpu.sync_copy(zbuf, table.at[pl.ds(off, ZCH)])

    pltpu.sync_copy(ys_h.at[pl.ds(s * 8, 8)], ysv)
    for j in range(8):
        for k in range(8):
            sl = pl.ds(j * 128 + k * L, L)
            y = ysv[j, pl.ds(k * L, L)]
            blo = jnp.where(y < SPLIT, 1, 0)
            bhi = jnp.where(y >= EDGE, 1, 0)
            inru = (1 - cb) * (blo + bhi) + cb * (1 - blo) * (1 - bhi)
            lr0 = y - bhi * (EDGE - TAILD)
            locr = (1 - cb) * lr0 + cb * (y - SPLIT)
            locg = locr * inru
            yloc[sl] = locg
            ylocb[sl] = locg + COLW
            ysct[j, pl.ds(k * L, L)] = locg + (1 - inru) * (DMY + iota16 + k * L)
            yval[sl] = inru.astype(jnp.float32)

    plsc.subcore_barrier()

    hh = [pltpu.async_copy(yval.at[pl.ds(j * 128, 128)],
                           table.at[ysct.at[j]], semh, add=True)
          for j in range(8)]
    for h in hh:
        h.wait()
    plsc.subcore_barrier()

    pltpu.sync_copy(table.at[yloc], cntv)

    def dz(i, carry):
        dacc[pl.ds(i * L, L)] = zero16
        return carry

    lax.fori_loop(0, B_T // L, dz, 0)

    for h in pending:
        h.wait()
    plsc.subcore_barrier()

    def gather(buf):
        idxb = yloc if buf == 0 else ylocb
        return pltpu.async_copy(colbuf.at[idxb], cgath, semg)

    def compute(buf):
        def grp(g, carry):
            sl = pl.ds(g * L, L)
            d = xst[buf, sl] - cgath[sl]
            dacc[sl] = dacc[sl] + d * d
            return carry

        lax.fori_loop(0, B_T // L, grp, 0)

    def drain1():
        pltpu.make_async_copy(cen_h.at[0].at[pl.ds(0, CS)],
                              colbuf.at[pl.ds(COLW, CS)], sems[1]).wait()
        pltpu.make_async_copy(xs_h.at[0].at[pl.ds(0, B_T)],
                              xst.at[1], sems[1]).wait()

    def feat_iter(kk, carry):
        g0 = gather(0)
        g0.wait()
        plsc.subcore_barrier()
        h0 = stage(jnp.minimum(2 * kk + 2, FEAT - 1), 0)
        compute(0)
        drain1()
        plsc.subcore_barrier()
        g1 = gather(1)
        g1.wait()
        plsc.subcore_barrier()
        stage(jnp.minimum(2 * kk + 3, FEAT - 1), 1)
        compute(1)
        for h in h0:
            h.wait()
        plsc.subcore_barrier()
        return carry

    lax.fori_loop(0, FEAT // 2, feat_iter, 0)
    drain1()

    def wsum(g, part):
        sl = pl.ds(g * L, L)
        cnt = jnp.maximum(cntv[sl], 1.0)
        return part + yval[sl] * dacc[sl] / (cnt * W_SCALE)

    part = lax.fori_loop(0, B_T // L, wsum, jnp.zeros((L,), jnp.float32))
    partv[...] = part
    pltpu.sync_copy(partv, out_h.at[wid])


@jax.jit
def kernel(xs, ys, center):
    ys2 = ys.reshape(BATCH // 128, 128)
    run = functools.partial(
        pl.kernel,
        mesh=plsc.VectorSubcoreMesh(core_axis_name="c", subcore_axis_name="s"),
        out_type=jax.ShapeDtypeStruct((NW, L), jnp.float32),
        scratch_types=[
            pltpu.VMEM_SHARED((TBL,), jnp.float32),
            pltpu.VMEM_SHARED((2 * COLW,), jnp.float32),
            pltpu.VMEM((ZCH,), jnp.float32),
            pltpu.VMEM((8, 128), jnp.int32),
            pltpu.VMEM((B_T,), jnp.int32),
            pltpu.VMEM((B_T,), jnp.int32),
            pltpu.VMEM((8, 128), jnp.int32),
            pltpu.VMEM((B_T,), jnp.float32),
            pltpu.VMEM((B_T,), jnp.float32),
            pltpu.VMEM((B_T,), jnp.float32),
            pltpu.VMEM((B_T,), jnp.float32),
            pltpu.VMEM((2, B_T), jnp.float32),
            pltpu.VMEM((64,), jnp.float32),
            pltpu.VMEM((L,), jnp.float32),
            pltpu.SemaphoreType.DMA,
            pltpu.SemaphoreType.DMA,
            pltpu.SemaphoreType.DMA,
            pltpu.SemaphoreType.DMA,
        ],
    )(_body)
    out = run(xs.T, ys2, center.T)
    return jnp.sum(out)


# speedup vs baseline: 1.5375x; 1.5375x over previous
import functools

import jax
import jax.numpy as jnp
from jax import lax
from jax.experimental import pallas as pl
from jax.experimental.pallas import tpu as pltpu
from jax.experimental.pallas import tpu_sc as plsc

CLS = 1_000_000
FEAT = 32
BATCH = 16384
NC = 2
NS = 16
L = 16
NW = NC * NS
B_T = BATCH // NS
SPLIT = 500_096
EDGE = 999_936
TAILD = 500_096
COLW = 500_224
CS = 31_232
CS_COVER = NS * CS
TBL = 500_736
REGION = TBL // NS
ZCH = REGION // 6
DMY = 500_160
W_SCALE = 2.0 * FEAT * CLS


def _body(xs_h, ys_h, cen_h, out_h,
          table, colbuf, zbuf, ysv, yloc, ylocb, ysct, yval,
          cntv, dacc, cgath, xst, tailv, partv, sem0, sem1, semg):
    c = lax.axis_index("c")
    s = lax.axis_index("s")
    wid = s * NC + c
    sems = [sem0, sem1]
    iota16 = lax.iota(jnp.int32, L)
    cb = jnp.full((L,), c)

    def stage(f, buf):
        boff = buf * COLW
        so = pl.multiple_of(c * SPLIT + s * CS, 128)
        do = pl.multiple_of(boff + s * CS, 8)
        hs = [
            pltpu.async_copy(cen_h.at[f].at[pl.ds(so, CS)],
                             colbuf.at[pl.ds(do, CS)], sems[buf]),
            pltpu.async_copy(xs_h.at[f].at[pl.ds(s * B_T, B_T)],
                             xst.at[buf], sems[buf]),
        ]

        @pl.when(s == 0)
        def _():
            @pl.when(c == 0)
            def _():
                pltpu.sync_copy(cen_h.at[f].at[pl.ds(CS_COVER, 384)],
                                colbuf.at[pl.ds(boff + CS_COVER, 384)])
                pltpu.sync_copy(cen_h.at[f].at[pl.ds(EDGE, 64)], tailv)
                pltpu.sync_copy(tailv, colbuf.at[pl.ds(boff + TAILD, 64)])

            @pl.when(c == 1)
            def _():
                pltpu.sync_copy(cen_h.at[f].at[pl.ds(SPLIT + CS_COVER, 128)],
                                colbuf.at[pl.ds(boff + CS_COVER, 128)])

        return hs

    pending = stage(0, 0)

    zero16 = jnp.zeros((L,), jnp.float32)

    def zfill(i, carry):
        zbuf[pl.ds(i * L, L)] = zero16
        return carry

    lax.fori_loop(0, ZCH // L, zfill, 0)
    for k in range(REGION // ZCH):
        off = pl.multiple_of(s * REGION + k * ZCH, 8)
        pltpu.sync_copy(zbuf, table.at[pl.ds(off, ZCH)])

    pltpu.sync_copy(ys_h.at[pl.ds(s * 8, 8)], ysv)
    for j in range(8):
        for k in range(8):
            sl = pl.ds(j * 128 + k * L, L)
            y = ysv[j, pl.ds(k * L, L)]
            blo = jnp.where(y < SPLIT, 1, 0)
            bhi = jnp.where(y >= EDGE, 1, 0)
            inru = (1 - cb) * (blo + bhi) + cb * (1 - blo) * (1 - bhi)
            lr0 = y - bhi * (EDGE - TAILD)
            locr = (1 - cb) * lr0 + cb * (y - SPLIT)
            locg = locr * inru
            yloc[sl] = locg
            ylocb[sl] = locg + COLW
            ysct[j, pl.ds(k * L, L)] = locg + (1 - inru) * (DMY + iota16 + k * L)
            yval[sl] = inru.astype(jnp.float32)

    plsc.subcore_barrier()

    for j in range(8):
        pltpu.sync_copy(yval.at[pl.ds(j * 128, 128)],
                        table.at[ysct.at[j]], add=True)
    plsc.subcore_barrier()

    pltpu.sync_copy(table.at[yloc], cntv)

    def dz(i, carry):
        dacc[pl.ds(i * L, L)] = zero16
        return carry

    lax.fori_loop(0, B_T // L, dz, 0)

    for h in pending:
        h.wait()
    plsc.subcore_barrier()

    H = B_T // 2

    def gather(buf, half):
        idxb = yloc if buf == 0 else ylocb
        return pltpu.async_copy(colbuf.at[idxb.at[pl.ds(half * H, H)]],
                                cgath.at[pl.ds(half * H, H)], semg)

    def compute(buf, half):
        def grp(g, carry):
            sl = pl.ds(half * H + g * L, L)
            d = xst[buf, sl] - cgath[sl]
            dacc[sl] = dacc[sl] + d * d
            return carry

        lax.fori_loop(0, H // L, grp, 0)

    def proc(buf):
        ga = gather(buf, 0)
        gb = gather(buf, 1)
        ga.wait()
        compute(buf, 0)
        gb.wait()
        compute(buf, 1)

    def feat_iter(kk, carry):
        h1 = stage(2 * kk + 1, 1)
        proc(0)
        for h in h1:
            h.wait()
        plsc.subcore_barrier()
        h0 = stage(jnp.minimum(2 * kk + 2, FEAT - 1), 0)
        proc(1)
        for h in h0:
            h.wait()
        plsc.subcore_barrier()
        return carry

    lax.fori_loop(0, FEAT // 2, feat_iter, 0)

    def wsum(g, part):
        sl = pl.ds(g * L, L)
        cnt = jnp.maximum(cntv[sl], 1.0)
        return part + yval[sl] * dacc[sl] / (cnt * W_SCALE)

    part = lax.fori_loop(0, B_T // L, wsum, jnp.zeros((L,), jnp.float32))
    partv[...] = part
    pltpu.sync_copy(partv, out_h.at[wid])


@jax.jit
def kernel(xs, ys, center):
    ys2 = ys.reshape(BATCH // 128, 128)
    run = functools.partial(
        pl.kernel,
        mesh=plsc.VectorSubcoreMesh(core_axis_name="c", subcore_axis_name="s"),
        out_type=jax.ShapeDtypeStruct((NW, L), jnp.float32),
        scratch_types=[
            pltpu.VMEM_SHARED((TBL,), jnp.float32),
            pltpu.VMEM_SHARED((2 * COLW,), jnp.float32),
            pltpu.VMEM((ZCH,), jnp.float32),
            pltpu.VMEM((8, 128), jnp.int32),
            pltpu.VMEM((B_T,), jnp.int32),
            pltpu.VMEM((B_T,), jnp.int32),
            pltpu.VMEM((8, 128), jnp.int32),
            pltpu.VMEM((B_T,), jnp.float32),
            pltpu.VMEM((B_T,), jnp.float32),
            pltpu.VMEM((B_T,), jnp.float32),
            pltpu.VMEM((B_T,), jnp.float32),
            pltpu.VMEM((2, B_T), jnp.float32),
            pltpu.VMEM((64,), jnp.float32),
            pltpu.VMEM((L,), jnp.float32),
            pltpu.SemaphoreType.DMA,
            pltpu.SemaphoreType.DMA,
            pltpu.SemaphoreType.DMA,
        ],
    )(_body)
    out = run(xs.T, ys2, center.T)
    return jnp.sum(out)


# speedup vs baseline: 1.5477x; 1.0066x over previous
import functools

import jax
import jax.numpy as jnp
from jax import lax
from jax.experimental import pallas as pl
from jax.experimental.pallas import tpu as pltpu
from jax.experimental.pallas import tpu_sc as plsc

CLS = 1_000_000
FEAT = 32
BATCH = 16384
NC = 2
NS = 16
L = 16
NW = NC * NS
B_T = BATCH // NS
SPLIT = 500_096
EDGE = 999_936
TAILD = 500_096
COLW = 500_224
CS = 31_232
CS_COVER = NS * CS
TBL = 500_736
REGION = TBL // NS
ZCH = REGION // 6
DMY = 500_160
W_SCALE = 2.0 * FEAT * CLS


def _body(xs_h, ys_h, cen_h, out_h,
          table, colbuf, zbuf, ysv, yloc, ylocb, ysct, yval,
          cntv, dacc, cgath, xst, tailv, partv, sem0, sem1, semg, semh):
    c = lax.axis_index("c")
    s = lax.axis_index("s")
    wid = s * NC + c
    sems = [sem0, sem1]
    iota16 = lax.iota(jnp.int32, L)
    cb = jnp.full((L,), c)

    def stage(f, buf):
        boff = buf * COLW
        so = pl.multiple_of(c * SPLIT + s * CS, 128)
        do = pl.multiple_of(boff + s * CS, 8)
        hs = [
            pltpu.async_copy(cen_h.at[f].at[pl.ds(so, CS)],
                             colbuf.at[pl.ds(do, CS)], sems[buf]),
            pltpu.async_copy(xs_h.at[f].at[pl.ds(s * B_T, B_T)],
                             xst.at[buf], sems[buf]),
        ]

        @pl.when((s == 1) & (c == 0))
        def _():
            pltpu.sync_copy(cen_h.at[f].at[pl.ds(CS_COVER, 384)],
                            colbuf.at[pl.ds(boff + CS_COVER, 384)])

        @pl.when((s == 2) & (c == 0))
        def _():
            pltpu.sync_copy(cen_h.at[f].at[pl.ds(EDGE, 64)], tailv)
            pltpu.sync_copy(tailv, colbuf.at[pl.ds(boff + TAILD, 64)])

        @pl.when((s == 3) & (c == 1))
        def _():
            pltpu.sync_copy(cen_h.at[f].at[pl.ds(SPLIT + CS_COVER, 128)],
                            colbuf.at[pl.ds(boff + CS_COVER, 128)])

        return hs

    pending = stage(0, 0)

    zero16 = jnp.zeros((L,), jnp.float32)

    def zfill(i, carry):
        zbuf[pl.ds(i * L, L)] = zero16
        return carry

    lax.fori_loop(0, ZCH // L, zfill, 0)
    for k in range(REGION // ZCH):
        off = pl.multiple_of(s * REGION + k * ZCH, 8)
        pltpu.sync_copy(zbuf, table.at[pl.ds(off, ZCH)])

    pltpu.sync_copy(ys_h.at[pl.ds(s * 8, 8)], ysv)
    for j in range(8):
        for k in range(8):
            sl = pl.ds(j * 128 + k * L, L)
            y = ysv[j, pl.ds(k * L, L)]
            blo = jnp.where(y < SPLIT, 1, 0)
            bhi = jnp.where(y >= EDGE, 1, 0)
            inru = (1 - cb) * (blo + bhi) + cb * (1 - blo) * (1 - bhi)
            lr0 = y - bhi * (EDGE - TAILD)
            locr = (1 - cb) * lr0 + cb * (y - SPLIT)
            locg = locr * inru
            yloc[sl] = locg
            ylocb[sl] = locg + COLW
            ysct[j, pl.ds(k * L, L)] = locg + (1 - inru) * (DMY + iota16 + k * L)
            yval[sl] = inru.astype(jnp.float32)

    plsc.subcore_barrier()

    hh = [pltpu.async_copy(yval.at[pl.ds(j * 128, 128)],
                           table.at[ysct.at[j]], semh, add=True)
          for j in range(8)]
    for h in hh:
        h.wait()
    plsc.subcore_barrier()

    pltpu.sync_copy(table.at[yloc], cntv)

    def dz(i, carry):
        dacc[pl.ds(i * L, L)] = zero16
        return carry

    lax.fori_loop(0, B_T // L, dz, 0)

    for h in pending:
        h.wait()
    plsc.subcore_barrier()

    def gather(buf):
        idxb = yloc if buf == 0 else ylocb
        return pltpu.async_copy(colbuf.at[idxb], cgath, semg)

    def compute(buf):
        def grp(g, carry):
            sl = pl.ds(g * L, L)
            d = xst[buf, sl] - cgath[sl]
            dacc[sl] = dacc[sl] + d * d
            return carry

        lax.fori_loop(0, B_T // L, grp, 0)

    def feat_iter(kk, carry):
        g0 = gather(0)
        h1 = stage(2 * kk + 1, 1)
        g0.wait()
        compute(0)
        for h in h1:
            h.wait()
        plsc.subcore_barrier()
        g1 = gather(1)
        h0 = stage(jnp.minimum(2 * kk + 2, FEAT - 1), 0)
        g1.wait()
        compute(1)
        for h in h0:
            h.wait()
        plsc.subcore_barrier()
        return carry

    lax.fori_loop(0, FEAT // 2, feat_iter, 0)

    def wsum(g, part):
        sl = pl.ds(g * L, L)
        cnt = jnp.maximum(cntv[sl], 1.0)
        return part + yval[sl] * dacc[sl] / (cnt * W_SCALE)

    part = lax.fori_loop(0, B_T // L, wsum, jnp.zeros((L,), jnp.float32))
    partv[...] = part
    pltpu.sync_copy(partv, out_h.at[wid])


@jax.jit
def kernel(xs, ys, center):
    ys2 = ys.reshape(BATCH // 128, 128)
    run = functools.partial(
        pl.kernel,
        mesh=plsc.VectorSubcoreMesh(core_axis_name="c", subcore_axis_name="s"),
        out_type=jax.ShapeDtypeStruct((NW, L), jnp.float32),
        scratch_types=[
            pltpu.VMEM_SHARED((TBL,), jnp.float32),
            pltpu.VMEM_SHARED((2 * COLW,), jnp.float32),
            pltpu.VMEM((ZCH,), jnp.float32),
            pltpu.VMEM((8, 128), jnp.int32),
            pltpu.VMEM((B_T,), jnp.int32),
            pltpu.VMEM((B_T,), jnp.int32),
            pltpu.VMEM((8, 128), jnp.int32),
            pltpu.VMEM((B_T,), jnp.float32),
            pltpu.VMEM((B_T,), jnp.float32),
            pltpu.VMEM((B_T,), jnp.float32),
            pltpu.VMEM((B_T,), jnp.float32),
            pltpu.VMEM((2, B_T), jnp.float32),
            pltpu.VMEM((64,), jnp.float32),
            pltpu.VMEM((L,), jnp.float32),
            pltpu.SemaphoreType.DMA,
            pltpu.SemaphoreType.DMA,
            pltpu.SemaphoreType.DMA,
            pltpu.SemaphoreType.DMA,
        ],
    )(_body)
    out = run(xs.T, ys2, center.T)
    return jnp.sum(out)
